# baseline (speedup 1.0000x reference)
import functools

import jax
import jax.numpy as jnp
from jax import lax
from jax.experimental import pallas as pl
from jax.experimental.pallas import tpu as pltpu
from jax.experimental.pallas import tpu_sc as plsc

B = 4
C = 8
S = 4096
V = 2048
H = 1024

NC = 2
NS = 16
NW = NC * NS

ROWS = B * S
RPW = ROWS // NW
NR = 2
NCHUNK = RPW // NR
GROWS = NR * C
NBUF = 6
LANES = 16


def _emb_body(tab, ids, out, idx_v, *bufs):
    wid = lax.axis_index("s") * NC + lax.axis_index("c")
    base = wid * RPW

    gbufs = bufs[0:NBUF]
    obufs = bufs[NBUF:2 * NBUF]
    gsems = bufs[2 * NBUF:3 * NBUF]
    osems = bufs[3 * NBUF:4 * NBUF]

    pltpu.sync_copy(ids.at[pl.ds(base * C, RPW * C)], idx_v)

    lane = lax.iota(jnp.int32, 16)
    off = (lane & (C - 1)) * V

    @plsc.parallel_loop(0, (RPW * C) // LANES, unroll=4)
    def add_off(j):
        sl = pl.ds(j * LANES, LANES)
        idx_v[sl] = idx_v[sl] + off

    def gather(c, slot):
        return pltpu.make_async_copy(
            tab.at[idx_v.at[pl.ds(c * GROWS, GROWS)]], gbufs[slot], gsems[slot]
        )

    def out_copy(c, slot):
        return pltpu.make_async_copy(
            obufs[slot], out.at[pl.ds(base + c * NR, NR)], osems[slot]
        )

    for b in range(NBUF):
        gather(b, b).start()

    def chunk_step(c, slot):
        gather(c, slot).wait()

        @pl.when(c >= NBUF)
        def _():
            out_copy(c - NBUF, slot).wait()

        gb = gbufs[slot]
        ob = obufs[slot]
        for r in range(NR):
            @plsc.parallel_loop(0, H // LANES, unroll=4)
            def acc_body(h):
                sl = pl.ds(h * LANES, LANES)
                acc = gb[r * C, sl]
                for k in range(1, C):
                    acc = acc + gb[r * C + k, sl]
                ob[r, sl] = acc

        out_copy(c, slot).start()

        @pl.when(c + NBUF < NCHUNK)
        def _():
            gather(c + NBUF, slot).start()

    def loop_body(g, carry):
        for b in range(NBUF):
            chunk_step(g * NBUF + b, b)
        return carry

    main = (NCHUNK // NBUF) * NBUF
    lax.fori_loop(0, NCHUNK // NBUF, loop_body, None)
    for c in range(main, NCHUNK):
        chunk_step(c, c % NBUF)

    for c in range(NCHUNK - NBUF, NCHUNK):
        out_copy(c, c % NBUF).wait()


_emb_kernel = functools.partial(
    pl.kernel,
    mesh=plsc.VectorSubcoreMesh(core_axis_name="c", subcore_axis_name="s"),
    out_type=jax.ShapeDtypeStruct((ROWS, H), jnp.float32),
    scratch_types=(
        [pltpu.VMEM((RPW * C,), jnp.int32)]
        + [pltpu.VMEM((GROWS, H), jnp.float32) for _ in range(NBUF)]
        + [pltpu.VMEM((NR, H), jnp.float32) for _ in range(NBUF)]
        + [pltpu.SemaphoreType.DMA for _ in range(2 * NBUF)]
    ),
)(_emb_body)


@jax.jit
def kernel(input_ids, tables):
    ids = input_ids.astype(jnp.int32).transpose(0, 2, 1).reshape(-1)
    tab = tables.reshape(C * V, H)
    out = _emb_kernel(tab, ids)
    return out.reshape(B, S, H)


# speedup vs baseline: 1.0324x; 1.0324x over previous
import functools

import jax
import jax.numpy as jnp
from jax import lax
from jax.experimental import pallas as pl
from jax.experimental.pallas import tpu as pltpu
from jax.experimental.pallas import tpu_sc as plsc

B = 4
C = 8
S = 4096
V = 2048
H = 1024

NC = 2
NS = 16
NW = NC * NS

ROWS = B * S
RPW = ROWS // NW
NR = 2
NCHUNK = RPW // NR
GROWS = NR * C
NBUF = 4
LANES = 16


def _emb_body(tab, ids, out, idx_v, *bufs):
    wid = lax.axis_index("s") * NC + lax.axis_index("c")
    base = wid * RPW

    gbufs = bufs[0:NBUF]
    obufs = bufs[NBUF:2 * NBUF]
    gsems = bufs[2 * NBUF:3 * NBUF]
    osems = bufs[3 * NBUF:4 * NBUF]

    pltpu.sync_copy(ids.at[pl.ds(base * C, RPW * C)], idx_v)

    lane = lax.iota(jnp.int32, 16)
    off = (lane & (C - 1)) * V

    @plsc.parallel_loop(0, (RPW * C) // LANES, unroll=4)
    def add_off(j):
        sl = pl.ds(j * LANES, LANES)
        idx_v[sl] = idx_v[sl] + off

    def gather(c, slot):
        return pltpu.make_async_copy(
            tab.at[idx_v.at[pl.ds(c * GROWS, GROWS)]], gbufs[slot], gsems[slot]
        )

    def out_copy(c, slot):
        return pltpu.make_async_copy(
            obufs[slot], out.at[pl.ds(base + c * NR, NR)], osems[slot]
        )

    for b in range(NBUF):
        gather(b, b).start()

    def chunk_step(c, slot):
        gather(c, slot).wait()

        @pl.when(c >= NBUF)
        def _():
            out_copy(c - NBUF, slot).wait()

        gb = gbufs[slot]
        ob = obufs[slot]
        for r in range(NR):
            @plsc.parallel_loop(0, H // LANES, unroll=4)
            def acc_body(h):
                sl = pl.ds(h * LANES, LANES)
                acc = gb[r * C, sl]
                for k in range(1, C):
                    acc = acc + gb[r * C + k, sl]
                ob[r, sl] = acc

        out_copy(c, slot).start()

        @pl.when(c + NBUF < NCHUNK)
        def _():
            gather(c + NBUF, slot).start()

    def loop_body(g, carry):
        for b in range(NBUF):
            chunk_step(g * NBUF + b, b)
        return carry

    main = (NCHUNK // NBUF) * NBUF
    lax.fori_loop(0, NCHUNK // NBUF, loop_body, None)
    for c in range(main, NCHUNK):
        chunk_step(c, c % NBUF)

    for c in range(NCHUNK - NBUF, NCHUNK):
        out_copy(c, c % NBUF).wait()


_emb_kernel = functools.partial(
    pl.kernel,
    mesh=plsc.VectorSubcoreMesh(core_axis_name="c", subcore_axis_name="s"),
    out_type=jax.ShapeDtypeStruct((ROWS, H), jnp.float32),
    scratch_types=(
        [pltpu.VMEM((RPW * C,), jnp.int32)]
        + [pltpu.VMEM((GROWS, H), jnp.float32) for _ in range(NBUF)]
        + [pltpu.VMEM((NR, H), jnp.float32) for _ in range(NBUF)]
        + [pltpu.SemaphoreType.DMA for _ in range(2 * NBUF)]
    ),
)(_emb_body)


@jax.jit
def kernel(input_ids, tables):
    ids = input_ids.astype(jnp.int32).transpose(0, 2, 1).reshape(-1)
    tab = tables.reshape(C * V, H)
    out = _emb_kernel(tab, ids)
    return out.reshape(B, S, H)
